# Initial kernel scaffold; baseline (speedup 1.0000x reference)
import jax
import jax.numpy as jnp
from jax.experimental import pallas as pl


def kernel(data, region_prototypes):
    raise NotImplementedError("write your pallas kernel here")


# speedup vs baseline: 4.3516x; 4.3516x over previous
import functools

import jax
import jax.numpy as jnp
from jax.experimental import pallas as pl

RAW = 512
REG = 32
GATHER = 16
BB = 8


def _body(x_ref, p_ref, o_ref):
    x = x_ref[...]
    o_ref[:, :RAW, :] = x[:, :RAW, :]
    xr = x[:, :GATHER, :]
    p = p_ref[0]
    sim = jnp.sum(xr * p[None, None, :], axis=2, keepdims=True) / 16.0
    m = jnp.max(sim, axis=1, keepdims=True)
    e = jnp.exp(sim - m)
    attn = e / jnp.sum(e, axis=1, keepdims=True)
    feat = jnp.sum(attn * xr, axis=1, keepdims=True)
    o_ref[:, RAW : RAW + 1, :] = feat
    o_ref[:, RAW + 1 :, :] = jnp.zeros_like(o_ref[:, RAW + 1 :, :])


@jax.jit
def kernel(data, region_prototypes):
    B, T, C = data.shape
    grid = (B // BB,)
    return pl.pallas_call(
        _body,
        grid=grid,
        in_specs=[
            pl.BlockSpec((BB, T, C), lambda b: (b, 0, 0)),
            pl.BlockSpec((REG, C), lambda b: (0, 0)),
        ],
        out_specs=pl.BlockSpec((BB, T, C), lambda b: (b, 0, 0)),
        out_shape=jax.ShapeDtypeStruct((B, T, C), data.dtype),
    )(data, region_prototypes)
